# Initial kernel scaffold; baseline (speedup 1.0000x reference)
import jax
import jax.numpy as jnp
from jax.experimental import pallas as pl


def kernel(x, weight_quant_packed, c):
    raise NotImplementedError("write your pallas kernel here")


# speedup vs baseline: 2.4251x; 2.4251x over previous
import functools

import jax
import jax.numpy as jnp
from jax import lax
from jax.experimental import pallas as pl
from jax.experimental.pallas import tpu as pltpu
from jax.experimental.pallas import tpu_sc as plsc

NUM_EMBEDDINGS = 100000
PACKED_WORDS = 16
EMB_DIM = 128
ROWS = 16384 * 26
NC, NS, L = 2, 16, 16
NW = NC * NS
ROWS_PER_W = ROWS // NW
CHUNK = 128
NCHUNK = ROWS_PER_W // CHUNK

SHIFTS = tuple(8 * (m // 2) + (4 if m % 2 == 0 else 0) for m in range(8))


def _make_kernel():
  mesh = plsc.VectorSubcoreMesh(core_axis_name="c", subcore_axis_name="s")

  @functools.partial(
      pl.kernel,
      mesh=mesh,
      out_type=jax.ShapeDtypeStruct((ROWS, EMB_DIM), jnp.float32),
      compiler_params=pltpu.CompilerParams(use_tc_tiling_on_sc=False),
      scratch_types=[
          pltpu.VMEM((CHUNK,), jnp.int32),
          pltpu.VMEM((CHUNK, PACKED_WORDS), jnp.int32),
          pltpu.VMEM((CHUNK, EMB_DIM), jnp.float32),
          pltpu.VMEM((L,), jnp.float32),
          pltpu.SemaphoreType.DMA,
      ],
  )
  def k(tab_hbm, idx_hbm, c_hbm, out_hbm, idx_v, rows_v, out_v, c_v, sem):
    wid = lax.axis_index("s") * NC + lax.axis_index("c")

    pltpu.sync_copy(c_hbm, c_v)
    c_vec = c_v[...]
    lut = (lax.iota(jnp.int32, L).astype(jnp.float32) - 7.0) / c_vec
    it = lax.iota(jnp.int32, L)
    m = it & 7
    shvec = (m >> 1) * 8 + (1 - (m & 1)) * 4
    wordsel = it >> 3

    def vperm(src, idx):
      return lax.gather(
          src, idx[:, None],
          lax.GatherDimensionNumbers(
              offset_dims=(), collapsed_slice_dims=(0,),
              start_index_map=(0,)),
          slice_sizes=(1,),
          mode=lax.GatherScatterMode.PROMISE_IN_BOUNDS)

    def chunk_body(g, _):
      base = wid * ROWS_PER_W + g * CHUNK
      pltpu.sync_copy(idx_hbm.at[pl.ds(base, CHUNK)], idx_v)
      pltpu.async_copy(tab_hbm.at[idx_v], rows_v, sem).wait()

      def row_body(i, _):
        w = rows_v[i, :]
        for s in range(8):
          wv = vperm(w, wordsel + 2 * s)
          nib = lax.shift_right_logical(wv, shvec) & 15
          out_v[i, pl.ds(s * L, L)] = vperm(lut, nib)
        return 0

      lax.fori_loop(0, CHUNK, row_body, 0)
      pltpu.sync_copy(out_v, out_hbm.at[pl.ds(base, CHUNK), :])
      return 0

    lax.fori_loop(0, NCHUNK, chunk_body, 0)

  return k


_sc_kernel = _make_kernel()


@jax.jit
def kernel(x, weight_quant_packed, c):
  tab32 = lax.bitcast_convert_type(
      weight_quant_packed.reshape(NUM_EMBEDDINGS, PACKED_WORDS, 4), jnp.int32)
  idx = x.reshape(ROWS)
  c_vec = jnp.full((L,), c, dtype=jnp.float32)
  out = _sc_kernel(tab32, idx, c_vec)
  return out.reshape(x.shape + (EMB_DIM,))


# speedup vs baseline: 3.0895x; 1.2740x over previous
import functools

import jax
import jax.numpy as jnp
from jax import lax
from jax.experimental import pallas as pl
from jax.experimental.pallas import tpu as pltpu
from jax.experimental.pallas import tpu_sc as plsc

NUM_EMBEDDINGS = 100000
PACKED_WORDS = 16
EMB_DIM = 128
ROWS = 16384 * 26
NC, NS, L = 2, 16, 16
NW = NC * NS
ROWS_PER_W = ROWS // NW
CHUNK = 128
NCHUNK = ROWS_PER_W // CHUNK


def _make_kernel():
  mesh = plsc.VectorSubcoreMesh(core_axis_name="c", subcore_axis_name="s")

  @functools.partial(
      pl.kernel,
      mesh=mesh,
      out_type=jax.ShapeDtypeStruct((ROWS, EMB_DIM), jnp.float32),
      compiler_params=pltpu.CompilerParams(use_tc_tiling_on_sc=False),
      scratch_types=[
          pltpu.VMEM((ROWS_PER_W,), jnp.int32),
          pltpu.VMEM((CHUNK, PACKED_WORDS), jnp.int32),
          pltpu.VMEM((CHUNK, PACKED_WORDS), jnp.int32),
          pltpu.VMEM((CHUNK, EMB_DIM), jnp.float32),
          pltpu.VMEM((CHUNK, EMB_DIM), jnp.float32),
          pltpu.VMEM((L,), jnp.float32),
          pltpu.SemaphoreType.DMA,
          pltpu.SemaphoreType.DMA,
          pltpu.SemaphoreType.DMA,
          pltpu.SemaphoreType.DMA,
      ],
  )
  def k(tab_hbm, idx_hbm, c_hbm, out_hbm, idx_all, rows0, rows1, out0, out1,
        c_v, sg0, sg1, so0, so1):
    wid = lax.axis_index("s") * NC + lax.axis_index("c")
    tbase = wid * ROWS_PER_W

    rows = (rows0, rows1)
    outs = (out0, out1)
    sg = (sg0, sg1)
    so = (so0, so1)

    pltpu.sync_copy(idx_hbm.at[pl.ds(tbase, ROWS_PER_W)], idx_all)
    pltpu.sync_copy(c_hbm, c_v)
    c_vec = c_v[...]
    lut = (lax.iota(jnp.int32, L).astype(jnp.float32) - 7.0) / c_vec
    it = lax.iota(jnp.int32, L)
    m = it & 7
    shvec = (m >> 1) * 8 + (1 - (m & 1)) * 4
    wordsel = it >> 3

    def vperm(src, idx):
      return lax.gather(
          src, idx[:, None],
          lax.GatherDimensionNumbers(
              offset_dims=(), collapsed_slice_dims=(0,),
              start_index_map=(0,)),
          slice_sizes=(1,),
          mode=lax.GatherScatterMode.PROMISE_IN_BOUNDS)

    def start_gather(g, b):
      pltpu.async_copy(
          tab_hbm.at[idx_all.at[pl.ds(g * CHUNK, CHUNK)]], rows[b], sg[b])

    def wait_gather(b):
      pltpu.make_async_copy(
          tab_hbm.at[pl.ds(0, CHUNK), :], rows[b], sg[b]).wait()

    def start_out(g, b):
      pltpu.async_copy(
          outs[b], out_hbm.at[pl.ds(tbase + g * CHUNK, CHUNK), :], so[b])

    def wait_out(b):
      pltpu.make_async_copy(
          outs[b], out_hbm.at[pl.ds(0, CHUNK), :], so[b]).wait()

    start_gather(0, 0)

    def chunk_pair(g2, _):
      for b in range(2):
        g = 2 * g2 + b
        nxt = g + 1

        @pl.when(nxt < NCHUNK)
        def _():
          start_gather(nxt, 1 - b)

        wait_gather(b)

        @pl.when(g >= 2)
        def _():
          wait_out(b)

        rows_b = rows[b]
        out_b = outs[b]

        def row_body(i, _):
          w = rows_b[i, :]
          for s in range(8):
            wv = vperm(w, wordsel + 2 * s)
            nib = lax.shift_right_logical(wv, shvec) & 15
            out_b[i, pl.ds(s * L, L)] = vperm(lut, nib)
          return 0

        lax.fori_loop(0, CHUNK, row_body, 0, unroll=2)
        start_out(g, b)
      return 0

    lax.fori_loop(0, NCHUNK // 2, chunk_pair, 0)
    wait_out(0)
    wait_out(1)

  return k


_sc_kernel = _make_kernel()


@jax.jit
def kernel(x, weight_quant_packed, c):
  tab32 = lax.bitcast_convert_type(
      weight_quant_packed.reshape(NUM_EMBEDDINGS, PACKED_WORDS, 4), jnp.int32)
  idx = x.reshape(ROWS)
  c_vec = jnp.full((L,), c, dtype=jnp.float32)
  out = _sc_kernel(tab32, idx, c_vec)
  return out.reshape(x.shape + (EMB_DIM,))
